# baseline (speedup 1.0000x reference)
import functools

import jax
import jax.numpy as jnp
from jax import lax
from jax.experimental import pallas as pl
from jax.experimental.pallas import tpu as pltpu
from jax.experimental.pallas import tpu_sc as plsc

N = 10000
D_IN = 128
H = 256
NH = 8
PHD = 32
NET = 4
NL = 4
INT = 512
NG = 64
OUT = 256

HH = H // 2
NP = 10112
NSUB = 16
WIN = 128
WPT = 320
CHW = 32
EPAD = NSUB * WPT * WIN
DWPT = 157
EPADD = 2 * NSUB * DWPT * WIN
RB = 1000
NBLK = N // RB

_MESH = dict(core_axis_name="c", subcore_axis_name="s")


def _sc_layer_body(tbl_h, gidx_h, dst_h, zeros_h, out_h, gv, dv, b0, b1, acc,
                   sg0, sg1, ss0, ss1):
    c = lax.axis_index("c")
    s = lax.axis_index("s")
    rows = NP // NSUB
    pltpu.sync_copy(zeros_h.at[pl.ds(s * rows, rows)],
                    acc.at[pl.ds(s * rows, rows)])
    plsc.subcore_barrier()
    tbl = tbl_h.at[c]
    gidx_t = gidx_h.at[s]
    dst_t = dst_h.at[s]

    @pl.loop(0, WPT // CHW)
    def _(ci):
        pltpu.sync_copy(gidx_t.at[pl.ds(ci * CHW, CHW)], gv)
        pltpu.sync_copy(dst_t.at[pl.ds(ci * CHW, CHW)], dv)
        pltpu.async_copy(tbl.at[gv.at[0]], b0, sg0)
        pltpu.async_copy(tbl.at[gv.at[1]], b1, sg1)

        @pl.loop(0, CHW, step=2)
        def _(j):
            pltpu.make_async_copy(tbl.at[gv.at[0]], b0, sg0).wait()
            pltpu.sync_copy(b0, acc.at[dv.at[j]], add=True)

            @pl.when(j + 2 < CHW)
            def _():
                pltpu.async_copy(tbl.at[gv.at[j + 2]], b0, sg0)

            pltpu.make_async_copy(tbl.at[gv.at[1]], b1, sg1).wait()
            pltpu.sync_copy(b1, acc.at[dv.at[j + 1]], add=True)

            @pl.when(j + 3 < CHW)
            def _():
                pltpu.async_copy(tbl.at[gv.at[j + 3]], b1, sg1)

    plsc.subcore_barrier()
    pltpu.sync_copy(acc.at[pl.ds(s * rows, rows)],
                    out_h.at[c].at[pl.ds(s * rows, rows)])


def _sc_layer(table, gidx, dstw, zeros):
    k = pl.kernel(
        _sc_layer_body,
        out_type=jax.ShapeDtypeStruct((2, NP, HH), jnp.float32),
        mesh=plsc.VectorSubcoreMesh(**_MESH),
        scratch_types=[
            pltpu.VMEM((CHW, WIN), jnp.int32),
            pltpu.VMEM((CHW, WIN), jnp.int32),
            pltpu.VMEM((WIN, HH), jnp.float32),
            pltpu.VMEM((WIN, HH), jnp.float32),
            pltpu.VMEM_SHARED((NP, HH), jnp.float32),
            pltpu.SemaphoreType.DMA,
            pltpu.SemaphoreType.DMA,
            pltpu.SemaphoreType.DMA,
            pltpu.SemaphoreType.DMA,
        ],
    )
    return k(table, gidx, dstw, zeros)


def _sc_deg_body(ddst_h, ones_h, zeros_h, out_h, dv, ones_v, acc):
    c = lax.axis_index("c")
    s = lax.axis_index("s")
    wid = c * NSUB + s
    pltpu.sync_copy(ddst_h.at[wid], dv)
    pltpu.sync_copy(ones_h, ones_v)
    rows = NP // NSUB
    pltpu.sync_copy(zeros_h.at[pl.ds(s * rows, rows)],
                    acc.at[pl.ds(s * rows, rows)])
    plsc.subcore_barrier()

    @pl.loop(0, DWPT)
    def _(j):
        pltpu.sync_copy(ones_v, acc.at[dv.at[j]], add=True)

    plsc.subcore_barrier()
    pltpu.sync_copy(acc.at[pl.ds(s * rows, rows)],
                    out_h.at[c].at[pl.ds(s * rows, rows)])


def _sc_deg(ddst, ones, zeros):
    k = pl.kernel(
        _sc_deg_body,
        out_type=jax.ShapeDtypeStruct((2, NP, HH), jnp.float32),
        mesh=plsc.VectorSubcoreMesh(**_MESH),
        scratch_types=[
            pltpu.VMEM((DWPT, WIN), jnp.int32),
            pltpu.VMEM((WIN, HH), jnp.float32),
            pltpu.VMEM_SHARED((NP, HH), jnp.float32),
        ],
    )
    return k(ddst, ones, zeros)


def _dot(a, b):
    return jnp.dot(a.astype(jnp.bfloat16), b.astype(jnp.bfloat16),
                   preferred_element_type=jnp.float32)


def _embed_body(x_ref, we_ref, wall_ref, h_ref, tab_ref):
    h = _dot(x_ref[...], we_ref[...])
    h_ref[...] = h
    tab_ref[0] = jax.nn.relu(_dot(h, wall_ref[0]))
    tab_ref[1] = jax.nn.relu(_dot(h, wall_ref[1]))


def _embed(x, W_embed, wall0):
    return pl.pallas_call(
        _embed_body,
        grid=(NBLK,),
        in_specs=[
            pl.BlockSpec((RB, D_IN), lambda i: (i, 0)),
            pl.BlockSpec((D_IN, H), lambda i: (0, 0)),
            pl.BlockSpec((2, H, NET * HH), lambda i: (0, 0, 0)),
        ],
        out_specs=[
            pl.BlockSpec((RB, H), lambda i: (i, 0)),
            pl.BlockSpec((2, RB, NET * HH), lambda i: (0, i, 0)),
        ],
        out_shape=[
            jax.ShapeDtypeStruct((N, H), jnp.float32),
            jax.ShapeDtypeStruct((2, N, NET * HH), jnp.float32),
        ],
    )(x, W_embed, wall0)


def _stats_body(degs_ref, inv_ref, samp_ref, satt_ref):
    dcol = degs_ref[0][:, 0:1] + degs_ref[1][:, 0:1]
    logd = jnp.log(dcol + 1.0)
    mask = lax.broadcasted_iota(jnp.int32, (NP, 1), 0) < N
    delta = jnp.sum(jnp.where(mask, logd, 0.0)) / N
    inv_ref[...] = 1.0 / jnp.maximum(dcol, 1.0)
    samp_ref[...] = logd / (delta + 1e-7)
    satt_ref[...] = delta / (logd + 1e-7)


def _stats(degs):
    return pl.pallas_call(
        _stats_body,
        grid=(1,),
        in_specs=[pl.BlockSpec((2, NP, HH), lambda i: (0, 0, 0))],
        out_specs=[pl.BlockSpec((NP, 1), lambda i: (0, 0))] * 3,
        out_shape=[jax.ShapeDtypeStruct((NP, 1), jnp.float32)] * 3,
    )(degs)


def _ln_rows(x, g, b):
    mu = jnp.mean(x, axis=-1, keepdims=True)
    var = jnp.mean((x - mu) * (x - mu), axis=-1, keepdims=True)
    return (x - mu) / jnp.sqrt(var + 1e-5) * g + b


def _post_body(emit_table, s0_ref, s1_ref, h_ref, inv_ref, samp_ref, satt_ref,
               A_ref, B_ref, C_ref, bout_ref, lg_ref, lb_ref, w1_ref, b1_ref,
               w2_ref, b2_ref, al_ref, *rest):
    if emit_table:
        wall_ref, hn_ref, tab_ref = rest
    else:
        (hn_ref,) = rest
    M = jnp.concatenate([s0_ref[...], s1_ref[...]], axis=1) * inv_ref[...]
    new = (_dot(M, A_ref[...]) + _dot(samp_ref[...] * M, B_ref[...])
           + _dot(satt_ref[...] * M, C_ref[...]) + bout_ref[...])
    al = al_ref[0, 0]
    h1 = h_ref[...] + al * new
    ln = _ln_rows(h1, lg_ref[...], lb_ref[...])
    boom = _dot(jax.nn.gelu(_dot(ln, w1_ref[...]) + b1_ref[...]),
                w2_ref[...]) + b2_ref[...]
    h2 = h1 + al * boom
    hn_ref[...] = h2
    if emit_table:
        tab_ref[0] = jax.nn.relu(_dot(h2, wall_ref[0]))
        tab_ref[1] = jax.nn.relu(_dot(h2, wall_ref[1]))


def _post(sum0, sum1, h, inv_deg, s_amp, s_att, A, B, C, bout, lg, lb,
          w1, b1, w2, b2, al, wall_next):
    emit_table = wall_next is not None
    in_specs = [
        pl.BlockSpec((RB, HH), lambda i: (i, 0)),
        pl.BlockSpec((RB, HH), lambda i: (i, 0)),
        pl.BlockSpec((RB, H), lambda i: (i, 0)),
        pl.BlockSpec((RB, 1), lambda i: (i, 0)),
        pl.BlockSpec((RB, 1), lambda i: (i, 0)),
        pl.BlockSpec((RB, 1), lambda i: (i, 0)),
        pl.BlockSpec((H, H), lambda i: (0, 0)),
        pl.BlockSpec((H, H), lambda i: (0, 0)),
        pl.BlockSpec((H, H), lambda i: (0, 0)),
        pl.BlockSpec((1, H), lambda i: (0, 0)),
        pl.BlockSpec((1, H), lambda i: (0, 0)),
        pl.BlockSpec((1, H), lambda i: (0, 0)),
        pl.BlockSpec((H, INT), lambda i: (0, 0)),
        pl.BlockSpec((1, INT), lambda i: (0, 0)),
        pl.BlockSpec((INT, H), lambda i: (0, 0)),
        pl.BlockSpec((1, H), lambda i: (0, 0)),
        pl.BlockSpec((1, 1), lambda i: (0, 0)),
    ]
    out_specs = [pl.BlockSpec((RB, H), lambda i: (i, 0))]
    out_shape = [jax.ShapeDtypeStruct((N, H), jnp.float32)]
    args = [sum0, sum1, h, inv_deg, s_amp, s_att, A, B, C, bout, lg, lb,
            w1, b1, w2, b2, al]
    if emit_table:
        in_specs.append(pl.BlockSpec((2, H, NET * HH), lambda i: (0, 0, 0)))
        out_specs.append(pl.BlockSpec((2, RB, NET * HH), lambda i: (0, i, 0)))
        out_shape.append(jax.ShapeDtypeStruct((2, N, NET * HH), jnp.float32))
        args.append(wall_next)
    res = pl.pallas_call(
        functools.partial(_post_body, emit_table),
        grid=(NBLK,),
        in_specs=in_specs,
        out_specs=out_specs,
        out_shape=out_shape,
    )(*args)
    return res if emit_table else (res[0], None)


def _read_body(h0_ref, h1_ref, h2_ref, h3_ref, h4_ref, b_ref, wr_ref, br_ref,
               fg_ref, fb_ref, out_ref, acc_ref, cnt_ref):
    i = pl.program_id(0)

    @pl.when(i == 0)
    def _():
        acc_ref[...] = jnp.zeros_like(acc_ref)
        cnt_ref[...] = jnp.zeros_like(cnt_ref)

    onehot = (b_ref[...] == lax.broadcasted_iota(jnp.int32, (RB, NG), 1))
    onehot = onehot.astype(jnp.float32)
    rep = jnp.concatenate([h0_ref[...], h1_ref[...], h2_ref[...],
                           h3_ref[...], h4_ref[...]], axis=1)
    dn = (((0,), (0,)), ((), ()))
    oh = onehot.astype(jnp.bfloat16)
    acc_ref[...] += lax.dot_general(oh, rep.astype(jnp.bfloat16), dn,
                                    preferred_element_type=jnp.float32)
    cnt_ref[...] += lax.dot_general(oh, jnp.ones((RB, 1), jnp.bfloat16),
                                    dn, preferred_element_type=jnp.float32)

    @pl.when(i == NBLK - 1)
    def _():
        gmean = acc_ref[...] / jnp.maximum(cnt_ref[...], 1.0)
        o = _dot(gmean, wr_ref[...]) + br_ref[...]
        out_ref[...] = _ln_rows(o, fg_ref[...], fb_ref[...])


def _readout(states, batch2d, W_read, b_read, fin_g, fin_b):
    return pl.pallas_call(
        _read_body,
        grid=(NBLK,),
        in_specs=[pl.BlockSpec((RB, H), lambda i: (i, 0))] * 5 + [
            pl.BlockSpec((RB, 1), lambda i: (i, 0)),
            pl.BlockSpec(((NL + 1) * H, OUT), lambda i: (0, 0)),
            pl.BlockSpec((1, OUT), lambda i: (0, 0)),
            pl.BlockSpec((1, OUT), lambda i: (0, 0)),
            pl.BlockSpec((1, OUT), lambda i: (0, 0)),
        ],
        out_specs=pl.BlockSpec((NG, OUT), lambda i: (0, 0)),
        out_shape=jax.ShapeDtypeStruct((NG, OUT), jnp.float32),
        scratch_shapes=[
            pltpu.VMEM((NG, (NL + 1) * H), jnp.float32),
            pltpu.VMEM((NG, 1), jnp.float32),
        ],
    )(*states, batch2d, W_read, b_read, fin_g, fin_b)


def kernel(x, edge_index, edge_attr, batch, num_graphs, W_embed, W_rel, Wout,
           bout, ln_g, ln_b, W1, b1, W2, b2, alpha, W_read, b_read, fin_g,
           fin_b):
    f32 = jnp.float32
    Wb = jnp.zeros((NL, NET, H, H), f32)
    for i in range(NH):
        Wb = Wb.at[:, :, i * PHD:(i + 1) * PHD, i * PHD:(i + 1) * PHD].set(
            W_rel[:, i])
    W_all = jnp.transpose(Wb, (0, 2, 1, 3))
    wall = jnp.stack([
        W_all[:, :, :, :HH].reshape(NL, H, NET * HH),
        W_all[:, :, :, HH:].reshape(NL, H, NET * HH),
    ], axis=1)
    Wr = Wout.reshape(NL, NH, 3, PHD, H)
    Amat = Wr[:, :, 0].reshape(NL, H, H)
    Bmat = Wr[:, :, 1].reshape(NL, H, H)
    Cmat = Wr[:, :, 2].reshape(NL, H, H)

    src = jnp.concatenate([edge_index[0], edge_index[1]])
    dst = jnp.concatenate([edge_index[1], edge_index[0]])
    ea2 = jnp.tile(edge_attr, (2,))
    e2 = src.shape[0]
    gidx = src * NET + ea2
    pad = EPAD - e2
    gidx_p = jnp.concatenate([gidx, jnp.zeros((pad,), jnp.int32)])
    dump = N + (jnp.arange(pad, dtype=jnp.int32) % 16)
    dst_p = jnp.concatenate([dst, dump])
    gidx_w = gidx_p.reshape(NSUB, WPT, WIN)
    dst_w = dst_p.reshape(NSUB, WPT, WIN)
    padd = EPADD - e2
    dumpd = N + (jnp.arange(padd, dtype=jnp.int32) % 16)
    ddst = jnp.concatenate([dst, dumpd]).reshape(2 * NSUB, DWPT, WIN)

    zeros_hh = jnp.zeros((NP, HH), f32)
    ones_win = jnp.ones((WIN, HH), f32)
    batch2d = batch.astype(jnp.int32).reshape(N, 1)

    degs = _sc_deg(ddst, ones_win, zeros_hh)
    inv_deg, s_amp, s_att = _stats(degs)

    h, table = _embed(x, W_embed, wall[0])

    states = [h]
    for l in range(NL):
        summed = _sc_layer(table.reshape(2, NET * N, HH), gidx_w, dst_w,
                           zeros_hh)
        wall_next = wall[l + 1] if l + 1 < NL else None
        h, table = _post(summed[0], summed[1], h, inv_deg, s_amp, s_att,
                         Amat[l], Bmat[l], Cmat[l],
                         bout[l].reshape(1, H), ln_g[l].reshape(1, H),
                         ln_b[l].reshape(1, H), W1[l], b1[l].reshape(1, INT),
                         W2[l], b2[l].reshape(1, H),
                         alpha[l].reshape(1, 1), wall_next)
        states.append(h)

    return _readout(states, batch2d, W_read, b_read.reshape(1, OUT),
                    fin_g.reshape(1, OUT), fin_b.reshape(1, OUT))


# speedup vs baseline: 1.0085x; 1.0085x over previous
import functools

import jax
import jax.numpy as jnp
from jax import lax
from jax.experimental import pallas as pl
from jax.experimental.pallas import tpu as pltpu
from jax.experimental.pallas import tpu_sc as plsc

N = 10000
D_IN = 128
H = 256
NH = 8
PHD = 32
NET = 4
NL = 4
INT = 512
NG = 64
OUT = 256

HH = H // 2
NP = 10112
NSUB = 16
WIN = 128
WPT = 320
CHW = 64
EPAD = NSUB * WPT * WIN
DWPT = 157
EPADD = 2 * NSUB * DWPT * WIN
RB = 1000
NBLK = N // RB

_MESH = dict(core_axis_name="c", subcore_axis_name="s")


def _sc_layer_body(tbl_h, gidx_h, dst_h, zeros_h, out_h, gv, dv, b0, b1, acc,
                   sg0, sg1, ss0, ss1):
    c = lax.axis_index("c")
    s = lax.axis_index("s")
    rows = NP // NSUB
    pltpu.sync_copy(zeros_h.at[pl.ds(s * rows, rows)],
                    acc.at[pl.ds(s * rows, rows)])
    plsc.subcore_barrier()
    tbl = tbl_h.at[c]
    gidx_t = gidx_h.at[s]
    dst_t = dst_h.at[s]

    @pl.loop(0, WPT // CHW)
    def _(ci):
        pltpu.sync_copy(gidx_t.at[pl.ds(ci * CHW, CHW)], gv)
        pltpu.sync_copy(dst_t.at[pl.ds(ci * CHW, CHW)], dv)
        pltpu.async_copy(tbl.at[gv.at[0]], b0, sg0)
        pltpu.async_copy(tbl.at[gv.at[1]], b1, sg1)

        @pl.loop(0, CHW, step=2)
        def _(j):
            pltpu.make_async_copy(tbl.at[gv.at[0]], b0, sg0).wait()
            pltpu.sync_copy(b0, acc.at[dv.at[j]], add=True)

            @pl.when(j + 2 < CHW)
            def _():
                pltpu.async_copy(tbl.at[gv.at[j + 2]], b0, sg0)

            pltpu.make_async_copy(tbl.at[gv.at[1]], b1, sg1).wait()
            pltpu.sync_copy(b1, acc.at[dv.at[j + 1]], add=True)

            @pl.when(j + 3 < CHW)
            def _():
                pltpu.async_copy(tbl.at[gv.at[j + 3]], b1, sg1)

    plsc.subcore_barrier()
    pltpu.sync_copy(acc.at[pl.ds(s * rows, rows)],
                    out_h.at[c].at[pl.ds(s * rows, rows)])


def _sc_layer(table, gidx, dstw, zeros):
    k = pl.kernel(
        _sc_layer_body,
        out_type=jax.ShapeDtypeStruct((2, NP, HH), jnp.float32),
        mesh=plsc.VectorSubcoreMesh(**_MESH),
        scratch_types=[
            pltpu.VMEM((CHW, WIN), jnp.int32),
            pltpu.VMEM((CHW, WIN), jnp.int32),
            pltpu.VMEM((WIN, HH), jnp.float32),
            pltpu.VMEM((WIN, HH), jnp.float32),
            pltpu.VMEM_SHARED((NP, HH), jnp.float32),
            pltpu.SemaphoreType.DMA,
            pltpu.SemaphoreType.DMA,
            pltpu.SemaphoreType.DMA,
            pltpu.SemaphoreType.DMA,
        ],
    )
    return k(table, gidx, dstw, zeros)


def _sc_deg_body(ddst_h, ones_h, zeros_h, out_h, dv, ones_v, acc):
    c = lax.axis_index("c")
    s = lax.axis_index("s")
    wid = c * NSUB + s
    pltpu.sync_copy(ddst_h.at[wid], dv)
    pltpu.sync_copy(ones_h, ones_v)
    rows = NP // NSUB
    pltpu.sync_copy(zeros_h.at[pl.ds(s * rows, rows)],
                    acc.at[pl.ds(s * rows, rows)])
    plsc.subcore_barrier()

    @pl.loop(0, DWPT)
    def _(j):
        pltpu.sync_copy(ones_v, acc.at[dv.at[j]], add=True)

    plsc.subcore_barrier()
    pltpu.sync_copy(acc.at[pl.ds(s * rows, rows)],
                    out_h.at[c].at[pl.ds(s * rows, rows)])


def _sc_deg(ddst, ones, zeros):
    k = pl.kernel(
        _sc_deg_body,
        out_type=jax.ShapeDtypeStruct((2, NP, HH), jnp.float32),
        mesh=plsc.VectorSubcoreMesh(**_MESH),
        scratch_types=[
            pltpu.VMEM((DWPT, WIN), jnp.int32),
            pltpu.VMEM((WIN, HH), jnp.float32),
            pltpu.VMEM_SHARED((NP, HH), jnp.float32),
        ],
    )
    return k(ddst, ones, zeros)


def _dot(a, b):
    return jnp.dot(a.astype(jnp.bfloat16), b.astype(jnp.bfloat16),
                   preferred_element_type=jnp.float32)


def _embed_body(x_ref, we_ref, wall_ref, h_ref, tab_ref):
    h = _dot(x_ref[...], we_ref[...])
    h_ref[...] = h
    tab_ref[0] = jax.nn.relu(_dot(h, wall_ref[0]))
    tab_ref[1] = jax.nn.relu(_dot(h, wall_ref[1]))


def _embed(x, W_embed, wall0):
    return pl.pallas_call(
        _embed_body,
        grid=(NBLK,),
        in_specs=[
            pl.BlockSpec((RB, D_IN), lambda i: (i, 0)),
            pl.BlockSpec((D_IN, H), lambda i: (0, 0)),
            pl.BlockSpec((2, H, NET * HH), lambda i: (0, 0, 0)),
        ],
        out_specs=[
            pl.BlockSpec((RB, H), lambda i: (i, 0)),
            pl.BlockSpec((2, RB, NET * HH), lambda i: (0, i, 0)),
        ],
        out_shape=[
            jax.ShapeDtypeStruct((N, H), jnp.float32),
            jax.ShapeDtypeStruct((2, N, NET * HH), jnp.float32),
        ],
    )(x, W_embed, wall0)


def _stats_body(degs_ref, inv_ref, samp_ref, satt_ref):
    dcol = degs_ref[0][:, 0:1] + degs_ref[1][:, 0:1]
    logd = jnp.log(dcol + 1.0)
    mask = lax.broadcasted_iota(jnp.int32, (NP, 1), 0) < N
    delta = jnp.sum(jnp.where(mask, logd, 0.0)) / N
    inv_ref[...] = 1.0 / jnp.maximum(dcol, 1.0)
    samp_ref[...] = logd / (delta + 1e-7)
    satt_ref[...] = delta / (logd + 1e-7)


def _stats(degs):
    return pl.pallas_call(
        _stats_body,
        grid=(1,),
        in_specs=[pl.BlockSpec((2, NP, HH), lambda i: (0, 0, 0))],
        out_specs=[pl.BlockSpec((NP, 1), lambda i: (0, 0))] * 3,
        out_shape=[jax.ShapeDtypeStruct((NP, 1), jnp.float32)] * 3,
    )(degs)


def _ln_rows(x, g, b):
    mu = jnp.mean(x, axis=-1, keepdims=True)
    var = jnp.mean((x - mu) * (x - mu), axis=-1, keepdims=True)
    return (x - mu) / jnp.sqrt(var + 1e-5) * g + b


def _post_body(emit_table, s0_ref, s1_ref, h_ref, inv_ref, samp_ref, satt_ref,
               A_ref, B_ref, C_ref, bout_ref, lg_ref, lb_ref, w1_ref, b1_ref,
               w2_ref, b2_ref, al_ref, *rest):
    if emit_table:
        wall_ref, hn_ref, tab_ref = rest
    else:
        (hn_ref,) = rest
    M = jnp.concatenate([s0_ref[...], s1_ref[...]], axis=1) * inv_ref[...]
    new = (_dot(M, A_ref[...]) + _dot(samp_ref[...] * M, B_ref[...])
           + _dot(satt_ref[...] * M, C_ref[...]) + bout_ref[...])
    al = al_ref[0, 0]
    h1 = h_ref[...] + al * new
    ln = _ln_rows(h1, lg_ref[...], lb_ref[...])
    boom = _dot(jax.nn.gelu(_dot(ln, w1_ref[...]) + b1_ref[...]),
                w2_ref[...]) + b2_ref[...]
    h2 = h1 + al * boom
    hn_ref[...] = h2
    if emit_table:
        tab_ref[0] = jax.nn.relu(_dot(h2, wall_ref[0]))
        tab_ref[1] = jax.nn.relu(_dot(h2, wall_ref[1]))


def _post(sum0, sum1, h, inv_deg, s_amp, s_att, A, B, C, bout, lg, lb,
          w1, b1, w2, b2, al, wall_next):
    emit_table = wall_next is not None
    in_specs = [
        pl.BlockSpec((RB, HH), lambda i: (i, 0)),
        pl.BlockSpec((RB, HH), lambda i: (i, 0)),
        pl.BlockSpec((RB, H), lambda i: (i, 0)),
        pl.BlockSpec((RB, 1), lambda i: (i, 0)),
        pl.BlockSpec((RB, 1), lambda i: (i, 0)),
        pl.BlockSpec((RB, 1), lambda i: (i, 0)),
        pl.BlockSpec((H, H), lambda i: (0, 0)),
        pl.BlockSpec((H, H), lambda i: (0, 0)),
        pl.BlockSpec((H, H), lambda i: (0, 0)),
        pl.BlockSpec((1, H), lambda i: (0, 0)),
        pl.BlockSpec((1, H), lambda i: (0, 0)),
        pl.BlockSpec((1, H), lambda i: (0, 0)),
        pl.BlockSpec((H, INT), lambda i: (0, 0)),
        pl.BlockSpec((1, INT), lambda i: (0, 0)),
        pl.BlockSpec((INT, H), lambda i: (0, 0)),
        pl.BlockSpec((1, H), lambda i: (0, 0)),
        pl.BlockSpec((1, 1), lambda i: (0, 0)),
    ]
    out_specs = [pl.BlockSpec((RB, H), lambda i: (i, 0))]
    out_shape = [jax.ShapeDtypeStruct((N, H), jnp.float32)]
    args = [sum0, sum1, h, inv_deg, s_amp, s_att, A, B, C, bout, lg, lb,
            w1, b1, w2, b2, al]
    if emit_table:
        in_specs.append(pl.BlockSpec((2, H, NET * HH), lambda i: (0, 0, 0)))
        out_specs.append(pl.BlockSpec((2, RB, NET * HH), lambda i: (0, i, 0)))
        out_shape.append(jax.ShapeDtypeStruct((2, N, NET * HH), jnp.float32))
        args.append(wall_next)
    res = pl.pallas_call(
        functools.partial(_post_body, emit_table),
        grid=(NBLK,),
        in_specs=in_specs,
        out_specs=out_specs,
        out_shape=out_shape,
    )(*args)
    return res if emit_table else (res[0], None)


def _read_body(h0_ref, h1_ref, h2_ref, h3_ref, h4_ref, b_ref, wr_ref, br_ref,
               fg_ref, fb_ref, out_ref, acc_ref, cnt_ref):
    i = pl.program_id(0)

    @pl.when(i == 0)
    def _():
        acc_ref[...] = jnp.zeros_like(acc_ref)
        cnt_ref[...] = jnp.zeros_like(cnt_ref)

    onehot = (b_ref[...] == lax.broadcasted_iota(jnp.int32, (RB, NG), 1))
    onehot = onehot.astype(jnp.float32)
    rep = jnp.concatenate([h0_ref[...], h1_ref[...], h2_ref[...],
                           h3_ref[...], h4_ref[...]], axis=1)
    dn = (((0,), (0,)), ((), ()))
    oh = onehot.astype(jnp.bfloat16)
    acc_ref[...] += lax.dot_general(oh, rep.astype(jnp.bfloat16), dn,
                                    preferred_element_type=jnp.float32)
    cnt_ref[...] += lax.dot_general(oh, jnp.ones((RB, 1), jnp.bfloat16),
                                    dn, preferred_element_type=jnp.float32)

    @pl.when(i == NBLK - 1)
    def _():
        gmean = acc_ref[...] / jnp.maximum(cnt_ref[...], 1.0)
        o = _dot(gmean, wr_ref[...]) + br_ref[...]
        out_ref[...] = _ln_rows(o, fg_ref[...], fb_ref[...])


def _readout(states, batch2d, W_read, b_read, fin_g, fin_b):
    return pl.pallas_call(
        _read_body,
        grid=(NBLK,),
        in_specs=[pl.BlockSpec((RB, H), lambda i: (i, 0))] * 5 + [
            pl.BlockSpec((RB, 1), lambda i: (i, 0)),
            pl.BlockSpec(((NL + 1) * H, OUT), lambda i: (0, 0)),
            pl.BlockSpec((1, OUT), lambda i: (0, 0)),
            pl.BlockSpec((1, OUT), lambda i: (0, 0)),
            pl.BlockSpec((1, OUT), lambda i: (0, 0)),
        ],
        out_specs=pl.BlockSpec((NG, OUT), lambda i: (0, 0)),
        out_shape=jax.ShapeDtypeStruct((NG, OUT), jnp.float32),
        scratch_shapes=[
            pltpu.VMEM((NG, (NL + 1) * H), jnp.float32),
            pltpu.VMEM((NG, 1), jnp.float32),
        ],
    )(*states, batch2d, W_read, b_read, fin_g, fin_b)


def kernel(x, edge_index, edge_attr, batch, num_graphs, W_embed, W_rel, Wout,
           bout, ln_g, ln_b, W1, b1, W2, b2, alpha, W_read, b_read, fin_g,
           fin_b):
    f32 = jnp.float32
    Wb = jnp.zeros((NL, NET, H, H), f32)
    for i in range(NH):
        Wb = Wb.at[:, :, i * PHD:(i + 1) * PHD, i * PHD:(i + 1) * PHD].set(
            W_rel[:, i])
    W_all = jnp.transpose(Wb, (0, 2, 1, 3))
    wall = jnp.stack([
        W_all[:, :, :, :HH].reshape(NL, H, NET * HH),
        W_all[:, :, :, HH:].reshape(NL, H, NET * HH),
    ], axis=1)
    Wr = Wout.reshape(NL, NH, 3, PHD, H)
    Amat = Wr[:, :, 0].reshape(NL, H, H)
    Bmat = Wr[:, :, 1].reshape(NL, H, H)
    Cmat = Wr[:, :, 2].reshape(NL, H, H)

    src = jnp.concatenate([edge_index[0], edge_index[1]])
    dst = jnp.concatenate([edge_index[1], edge_index[0]])
    ea2 = jnp.tile(edge_attr, (2,))
    e2 = src.shape[0]
    gidx = src * NET + ea2
    pad = EPAD - e2
    gidx_p = jnp.concatenate([gidx, jnp.zeros((pad,), jnp.int32)])
    dump = N + (jnp.arange(pad, dtype=jnp.int32) % 16)
    dst_p = jnp.concatenate([dst, dump])
    gidx_w = gidx_p.reshape(NSUB, WPT, WIN)
    dst_w = dst_p.reshape(NSUB, WPT, WIN)
    padd = EPADD - e2
    dumpd = N + (jnp.arange(padd, dtype=jnp.int32) % 16)
    ddst = jnp.concatenate([dst, dumpd]).reshape(2 * NSUB, DWPT, WIN)

    zeros_hh = jnp.zeros((NP, HH), f32)
    ones_win = jnp.ones((WIN, HH), f32)
    batch2d = batch.astype(jnp.int32).reshape(N, 1)

    degs = _sc_deg(ddst, ones_win, zeros_hh)
    inv_deg, s_amp, s_att = _stats(degs)

    h, table = _embed(x, W_embed, wall[0])

    states = [h]
    for l in range(NL):
        summed = _sc_layer(table.reshape(2, NET * N, HH), gidx_w, dst_w,
                           zeros_hh)
        wall_next = wall[l + 1] if l + 1 < NL else None
        h, table = _post(summed[0], summed[1], h, inv_deg, s_amp, s_att,
                         Amat[l], Bmat[l], Cmat[l],
                         bout[l].reshape(1, H), ln_g[l].reshape(1, H),
                         ln_b[l].reshape(1, H), W1[l], b1[l].reshape(1, INT),
                         W2[l], b2[l].reshape(1, H),
                         alpha[l].reshape(1, 1), wall_next)
        states.append(h)

    return _readout(states, batch2d, W_read, b_read.reshape(1, OUT),
                    fin_g.reshape(1, OUT), fin_b.reshape(1, OUT))


# speedup vs baseline: 1.0091x; 1.0006x over previous
import functools

import jax
import jax.numpy as jnp
from jax import lax
from jax.experimental import pallas as pl
from jax.experimental.pallas import tpu as pltpu
from jax.experimental.pallas import tpu_sc as plsc

N = 10000
D_IN = 128
H = 256
NH = 8
PHD = 32
NET = 4
NL = 4
INT = 512
NG = 64
OUT = 256

HH = H // 2
NP = 10112
NSUB = 16
WIN = 128
WPT = 320
CHW = 64
EPAD = NSUB * WPT * WIN
DWPT = 157
EPADD = 2 * NSUB * DWPT * WIN
RB = 1000
NBLK = N // RB

_MESH = dict(core_axis_name="c", subcore_axis_name="s")


def _sc_layer_body(tbl_h, gidx_h, dst_h, zeros_h, out_h, gv, dv, b0, b1, acc,
                   sg0, sg1):
    c = lax.axis_index("c")
    s = lax.axis_index("s")
    rows = NP // NSUB
    pltpu.sync_copy(zeros_h.at[pl.ds(s * rows, rows)],
                    acc.at[pl.ds(s * rows, rows)])
    plsc.subcore_barrier()
    tbl = tbl_h.at[c]
    gidx_t = gidx_h.at[s]
    dst_t = dst_h.at[s]

    @pl.loop(0, WPT // CHW)
    def _(ci):
        pltpu.sync_copy(gidx_t.at[pl.ds(ci * CHW, CHW)], gv)
        pltpu.sync_copy(dst_t.at[pl.ds(ci * CHW, CHW)], dv)
        pltpu.async_copy(tbl.at[gv.at[0]], b0, sg0)
        pltpu.async_copy(tbl.at[gv.at[1]], b1, sg1)

        @pl.loop(0, CHW, step=2)
        def _(j):
            pltpu.make_async_copy(tbl.at[gv.at[0]], b0, sg0).wait()
            pltpu.sync_copy(b0, acc.at[dv.at[j]], add=True)

            @pl.when(j + 2 < CHW)
            def _():
                pltpu.async_copy(tbl.at[gv.at[j + 2]], b0, sg0)

            pltpu.make_async_copy(tbl.at[gv.at[1]], b1, sg1).wait()
            pltpu.sync_copy(b1, acc.at[dv.at[j + 1]], add=True)

            @pl.when(j + 3 < CHW)
            def _():
                pltpu.async_copy(tbl.at[gv.at[j + 3]], b1, sg1)

    plsc.subcore_barrier()
    pltpu.sync_copy(acc.at[pl.ds(s * rows, rows)],
                    out_h.at[c].at[pl.ds(s * rows, rows)])


def _sc_layer(table, gidx, dstw, zeros):
    k = pl.kernel(
        _sc_layer_body,
        out_type=jax.ShapeDtypeStruct((2, NP, HH), jnp.float32),
        mesh=plsc.VectorSubcoreMesh(**_MESH),
        scratch_types=[
            pltpu.VMEM((CHW, WIN), jnp.int32),
            pltpu.VMEM((CHW, WIN), jnp.int32),
            pltpu.VMEM((WIN, HH), jnp.float32),
            pltpu.VMEM((WIN, HH), jnp.float32),
            pltpu.VMEM_SHARED((NP, HH), jnp.float32),
            pltpu.SemaphoreType.DMA,
            pltpu.SemaphoreType.DMA,
        ],
    )
    return k(table, gidx, dstw, zeros)


def _sc_deg_body(ddst_h, ones_h, zeros_h, out_h, dv, ones_v, acc):
    c = lax.axis_index("c")
    s = lax.axis_index("s")
    wid = c * NSUB + s
    pltpu.sync_copy(ddst_h.at[wid], dv)
    pltpu.sync_copy(ones_h, ones_v)
    rows = NP // NSUB
    pltpu.sync_copy(zeros_h.at[pl.ds(s * rows, rows)],
                    acc.at[pl.ds(s * rows, rows)])
    plsc.subcore_barrier()

    @pl.loop(0, DWPT)
    def _(j):
        pltpu.sync_copy(ones_v, acc.at[dv.at[j]], add=True)

    plsc.subcore_barrier()
    pltpu.sync_copy(acc.at[pl.ds(s * rows, rows)],
                    out_h.at[c].at[pl.ds(s * rows, rows)])


def _sc_deg(ddst, ones, zeros):
    k = pl.kernel(
        _sc_deg_body,
        out_type=jax.ShapeDtypeStruct((2, NP, HH), jnp.float32),
        mesh=plsc.VectorSubcoreMesh(**_MESH),
        scratch_types=[
            pltpu.VMEM((DWPT, WIN), jnp.int32),
            pltpu.VMEM((WIN, HH), jnp.float32),
            pltpu.VMEM_SHARED((NP, HH), jnp.float32),
        ],
    )
    return k(ddst, ones, zeros)


def _dot(a, b):
    return jnp.dot(a.astype(jnp.bfloat16), b.astype(jnp.bfloat16),
                   preferred_element_type=jnp.float32)


def _embed_body(x_ref, we_ref, wall_ref, h_ref, tab_ref):
    h = _dot(x_ref[...], we_ref[...])
    h_ref[...] = h
    tab_ref[0] = jax.nn.relu(_dot(h, wall_ref[0]))
    tab_ref[1] = jax.nn.relu(_dot(h, wall_ref[1]))


def _embed(x, W_embed, wall0):
    return pl.pallas_call(
        _embed_body,
        grid=(NBLK,),
        in_specs=[
            pl.BlockSpec((RB, D_IN), lambda i: (i, 0)),
            pl.BlockSpec((D_IN, H), lambda i: (0, 0)),
            pl.BlockSpec((2, H, NET * HH), lambda i: (0, 0, 0)),
        ],
        out_specs=[
            pl.BlockSpec((RB, H), lambda i: (i, 0)),
            pl.BlockSpec((2, RB, NET * HH), lambda i: (0, i, 0)),
        ],
        out_shape=[
            jax.ShapeDtypeStruct((N, H), jnp.float32),
            jax.ShapeDtypeStruct((2, N, NET * HH), jnp.float32),
        ],
    )(x, W_embed, wall0)


def _stats_body(degs_ref, inv_ref, samp_ref, satt_ref):
    dcol = degs_ref[0][:, 0:1] + degs_ref[1][:, 0:1]
    logd = jnp.log(dcol + 1.0)
    mask = lax.broadcasted_iota(jnp.int32, (NP, 1), 0) < N
    delta = jnp.sum(jnp.where(mask, logd, 0.0)) / N
    inv_ref[...] = 1.0 / jnp.maximum(dcol, 1.0)
    samp_ref[...] = logd / (delta + 1e-7)
    satt_ref[...] = delta / (logd + 1e-7)


def _stats(degs):
    return pl.pallas_call(
        _stats_body,
        grid=(1,),
        in_specs=[pl.BlockSpec((2, NP, HH), lambda i: (0, 0, 0))],
        out_specs=[pl.BlockSpec((NP, 1), lambda i: (0, 0))] * 3,
        out_shape=[jax.ShapeDtypeStruct((NP, 1), jnp.float32)] * 3,
    )(degs)


def _ln_rows(x, g, b):
    mu = jnp.mean(x, axis=-1, keepdims=True)
    var = jnp.mean((x - mu) * (x - mu), axis=-1, keepdims=True)
    return (x - mu) / jnp.sqrt(var + 1e-5) * g + b


def _post_body(emit_table, s0_ref, s1_ref, h_ref, inv_ref, samp_ref, satt_ref,
               A_ref, B_ref, C_ref, bout_ref, lg_ref, lb_ref, w1_ref, b1_ref,
               w2_ref, b2_ref, al_ref, *rest):
    if emit_table:
        wall_ref, hn_ref, tab_ref = rest
    else:
        (hn_ref,) = rest
    M = jnp.concatenate([s0_ref[...], s1_ref[...]], axis=1) * inv_ref[...]
    new = (_dot(M, A_ref[...]) + _dot(samp_ref[...] * M, B_ref[...])
           + _dot(satt_ref[...] * M, C_ref[...]) + bout_ref[...])
    al = al_ref[0, 0]
    h1 = h_ref[...] + al * new
    ln = _ln_rows(h1, lg_ref[...], lb_ref[...])
    boom = _dot(jax.nn.gelu(_dot(ln, w1_ref[...]) + b1_ref[...]),
                w2_ref[...]) + b2_ref[...]
    h2 = h1 + al * boom
    hn_ref[...] = h2
    if emit_table:
        tab_ref[0] = jax.nn.relu(_dot(h2, wall_ref[0]))
        tab_ref[1] = jax.nn.relu(_dot(h2, wall_ref[1]))


def _post(sum0, sum1, h, inv_deg, s_amp, s_att, A, B, C, bout, lg, lb,
          w1, b1, w2, b2, al, wall_next):
    emit_table = wall_next is not None
    in_specs = [
        pl.BlockSpec((RB, HH), lambda i: (i, 0)),
        pl.BlockSpec((RB, HH), lambda i: (i, 0)),
        pl.BlockSpec((RB, H), lambda i: (i, 0)),
        pl.BlockSpec((RB, 1), lambda i: (i, 0)),
        pl.BlockSpec((RB, 1), lambda i: (i, 0)),
        pl.BlockSpec((RB, 1), lambda i: (i, 0)),
        pl.BlockSpec((H, H), lambda i: (0, 0)),
        pl.BlockSpec((H, H), lambda i: (0, 0)),
        pl.BlockSpec((H, H), lambda i: (0, 0)),
        pl.BlockSpec((1, H), lambda i: (0, 0)),
        pl.BlockSpec((1, H), lambda i: (0, 0)),
        pl.BlockSpec((1, H), lambda i: (0, 0)),
        pl.BlockSpec((H, INT), lambda i: (0, 0)),
        pl.BlockSpec((1, INT), lambda i: (0, 0)),
        pl.BlockSpec((INT, H), lambda i: (0, 0)),
        pl.BlockSpec((1, H), lambda i: (0, 0)),
        pl.BlockSpec((1, 1), lambda i: (0, 0)),
    ]
    out_specs = [pl.BlockSpec((RB, H), lambda i: (i, 0))]
    out_shape = [jax.ShapeDtypeStruct((N, H), jnp.float32)]
    args = [sum0, sum1, h, inv_deg, s_amp, s_att, A, B, C, bout, lg, lb,
            w1, b1, w2, b2, al]
    if emit_table:
        in_specs.append(pl.BlockSpec((2, H, NET * HH), lambda i: (0, 0, 0)))
        out_specs.append(pl.BlockSpec((2, RB, NET * HH), lambda i: (0, i, 0)))
        out_shape.append(jax.ShapeDtypeStruct((2, N, NET * HH), jnp.float32))
        args.append(wall_next)
    res = pl.pallas_call(
        functools.partial(_post_body, emit_table),
        grid=(NBLK,),
        in_specs=in_specs,
        out_specs=out_specs,
        out_shape=out_shape,
    )(*args)
    return res if emit_table else (res[0], None)


def _read_body(h0_ref, h1_ref, h2_ref, h3_ref, h4_ref, b_ref, wr_ref, br_ref,
               fg_ref, fb_ref, out_ref, acc_ref, cnt_ref):
    i = pl.program_id(0)

    @pl.when(i == 0)
    def _():
        acc_ref[...] = jnp.zeros_like(acc_ref)
        cnt_ref[...] = jnp.zeros_like(cnt_ref)

    onehot = (b_ref[...] == lax.broadcasted_iota(jnp.int32, (RB, NG), 1))
    onehot = onehot.astype(jnp.float32)
    rep = jnp.concatenate([h0_ref[...], h1_ref[...], h2_ref[...],
                           h3_ref[...], h4_ref[...]], axis=1)
    dn = (((0,), (0,)), ((), ()))
    oh = onehot.astype(jnp.bfloat16)
    acc_ref[...] += lax.dot_general(oh, rep.astype(jnp.bfloat16), dn,
                                    preferred_element_type=jnp.float32)
    cnt_ref[...] += lax.dot_general(oh, jnp.ones((RB, 1), jnp.bfloat16),
                                    dn, preferred_element_type=jnp.float32)

    @pl.when(i == NBLK - 1)
    def _():
        gmean = acc_ref[...] / jnp.maximum(cnt_ref[...], 1.0)
        o = _dot(gmean, wr_ref[...]) + br_ref[...]
        out_ref[...] = _ln_rows(o, fg_ref[...], fb_ref[...])


def _readout(states, batch2d, W_read, b_read, fin_g, fin_b):
    return pl.pallas_call(
        _read_body,
        grid=(NBLK,),
        in_specs=[pl.BlockSpec((RB, H), lambda i: (i, 0))] * 5 + [
            pl.BlockSpec((RB, 1), lambda i: (i, 0)),
            pl.BlockSpec(((NL + 1) * H, OUT), lambda i: (0, 0)),
            pl.BlockSpec((1, OUT), lambda i: (0, 0)),
            pl.BlockSpec((1, OUT), lambda i: (0, 0)),
            pl.BlockSpec((1, OUT), lambda i: (0, 0)),
        ],
        out_specs=pl.BlockSpec((NG, OUT), lambda i: (0, 0)),
        out_shape=jax.ShapeDtypeStruct((NG, OUT), jnp.float32),
        scratch_shapes=[
            pltpu.VMEM((NG, (NL + 1) * H), jnp.float32),
            pltpu.VMEM((NG, 1), jnp.float32),
        ],
    )(*states, batch2d, W_read, b_read, fin_g, fin_b)


def kernel(x, edge_index, edge_attr, batch, num_graphs, W_embed, W_rel, Wout,
           bout, ln_g, ln_b, W1, b1, W2, b2, alpha, W_read, b_read, fin_g,
           fin_b):
    f32 = jnp.float32
    Wb = jnp.zeros((NL, NET, H, H), f32)
    for i in range(NH):
        Wb = Wb.at[:, :, i * PHD:(i + 1) * PHD, i * PHD:(i + 1) * PHD].set(
            W_rel[:, i])
    W_all = jnp.transpose(Wb, (0, 2, 1, 3))
    wall = jnp.stack([
        W_all[:, :, :, :HH].reshape(NL, H, NET * HH),
        W_all[:, :, :, HH:].reshape(NL, H, NET * HH),
    ], axis=1)
    Wr = Wout.reshape(NL, NH, 3, PHD, H)
    Amat = Wr[:, :, 0].reshape(NL, H, H)
    Bmat = Wr[:, :, 1].reshape(NL, H, H)
    Cmat = Wr[:, :, 2].reshape(NL, H, H)

    src = jnp.concatenate([edge_index[0], edge_index[1]])
    dst = jnp.concatenate([edge_index[1], edge_index[0]])
    ea2 = jnp.tile(edge_attr, (2,))
    e2 = src.shape[0]
    gidx = src * NET + ea2
    pad = EPAD - e2
    gidx_p = jnp.concatenate([gidx, jnp.zeros((pad,), jnp.int32)])
    dump = N + (jnp.arange(pad, dtype=jnp.int32) % 16)
    dst_p = jnp.concatenate([dst, dump])
    gidx_w = gidx_p.reshape(NSUB, WPT, WIN)
    dst_w = dst_p.reshape(NSUB, WPT, WIN)
    padd = EPADD - e2
    dumpd = N + (jnp.arange(padd, dtype=jnp.int32) % 16)
    ddst = jnp.concatenate([dst, dumpd]).reshape(2 * NSUB, DWPT, WIN)

    zeros_hh = jnp.zeros((NP, HH), f32)
    ones_win = jnp.ones((WIN, HH), f32)
    batch2d = batch.astype(jnp.int32).reshape(N, 1)

    degs = _sc_deg(ddst, ones_win, zeros_hh)
    inv_deg, s_amp, s_att = _stats(degs)

    h, table = _embed(x, W_embed, wall[0])

    states = [h]
    for l in range(NL):
        summed = _sc_layer(table.reshape(2, NET * N, HH), gidx_w, dst_w,
                           zeros_hh)
        wall_next = wall[l + 1] if l + 1 < NL else None
        h, table = _post(summed[0], summed[1], h, inv_deg, s_amp, s_att,
                         Amat[l], Bmat[l], Cmat[l],
                         bout[l].reshape(1, H), ln_g[l].reshape(1, H),
                         ln_b[l].reshape(1, H), W1[l], b1[l].reshape(1, INT),
                         W2[l], b2[l].reshape(1, H),
                         alpha[l].reshape(1, 1), wall_next)
        states.append(h)

    return _readout(states, batch2d, W_read, b_read.reshape(1, OUT),
                    fin_g.reshape(1, OUT), fin_b.reshape(1, OUT))
